# baseline (speedup 1.0000x reference)
import dataclasses
import functools

import jax
import jax.numpy as jnp
from jax import lax
from jax.experimental import pallas as pl
from jax.experimental.pallas import tpu as pltpu
from jax.experimental.pallas import tpu_sc as plsc

B = 16384
N = 100000
D = 256
K = 16
A = 64

NW = 32
CPW = B // NW
CHUNK = 32
NCHUNK = CPW // CHUNK
GB = 8
GROWS = GB * K
NBATCH = CHUNK // GB

_NEG = -1e30


def _tree_sum(vals):
    vals = list(vals)
    while len(vals) > 1:
        nxt = [vals[i] + vals[i + 1] for i in range(0, len(vals) - 1, 2)]
        if len(vals) % 2:
            nxt.append(vals[-1])
        vals = nxt
    return vals[0]


def _tc_pre_body(cb_ref, w_ref, wq_ref, q_ref, slog_ref):
    q_ref[...] = jnp.dot(cb_ref[...], wq_ref[...],
                         preferred_element_type=jnp.float32)
    w = w_ref[...]
    slog_ref[...] = jnp.where(w < 1e-6, _NEG, jnp.log(jnp.maximum(w, 1e-6)))


def _tc_keys_body(e_ref, wk_ref, k_ref):
    km = jnp.dot(e_ref[...], wk_ref[...], preferred_element_type=jnp.float32)
    k_ref[...] = jnp.concatenate(
        [km, jnp.zeros(km.shape, jnp.float32)], axis=1)


def _tc_post_body(cb_ref, wn_ref, wg1_ref, wg2_ref, bg_ref, o_ref):
    cb = cb_ref[...]
    wn = wn_ref[...]
    z = (jnp.dot(cb, wg1_ref[...], preferred_element_type=jnp.float32)
         + jnp.dot(wn, wg2_ref[...], preferred_element_type=jnp.float32)
         + bg_ref[...])
    g = jax.nn.sigmoid(z)
    o_ref[...] = g * cb + (1.0 - g) * wn


def _sc_attention(all_embs, keys, idx2, q, slog):
    mesh = plsc.VectorSubcoreMesh(core_axis_name="c", subcore_axis_name="s")
    cp = pltpu.CompilerParams()
    if "needs_layout_passes" in pltpu.CompilerParams.__dataclass_fields__:
        cp = dataclasses.replace(cp, needs_layout_passes=False)

    @functools.partial(
        pl.kernel,
        out_type=jax.ShapeDtypeStruct((B, D), jnp.float32),
        mesh=mesh,
        compiler_params=cp,
        scratch_types=[
            pltpu.VMEM((NBATCH, GROWS), jnp.int32),
            pltpu.VMEM((CHUNK, A), jnp.float32),
            pltpu.VMEM((CHUNK, K), jnp.float32),
            pltpu.VMEM((CHUNK, D), jnp.float32),
            pltpu.VMEM((GROWS, D), jnp.float32),
            pltpu.VMEM((GROWS, D), jnp.float32),
            pltpu.VMEM((GROWS, 2 * A), jnp.float32),
            pltpu.VMEM((GROWS, 2 * A), jnp.float32),
            pltpu.VMEM((2 * K, 16), jnp.float32),
            pltpu.SemaphoreType.DMA,
            pltpu.SemaphoreType.DMA,
            pltpu.SemaphoreType.DMA,
            pltpu.SemaphoreType.DMA,
        ],
    )
    def sc_kernel(embs_hbm, keys_hbm, idx_hbm, q_hbm, slog_hbm, wn_hbm,
                  idx_v, q_v, slog_v, out_v, bufA, bufB, kbufA, kbufB,
                  acc_v, semA, semB, semKA, semKB):
        cid = lax.axis_index("c")
        sid = lax.axis_index("s")
        wid = sid * 2 + cid
        lane = lax.broadcasted_iota(jnp.int32, (K,), 0)
        cols = [jnp.full((16,), d_, jnp.int32) for d_ in range(16)]

        def start_gathers(g, ebuf, kbuf, esem, ksem):
            pltpu.make_async_copy(embs_hbm.at[idx_v.at[g]], ebuf, esem).start()
            pltpu.make_async_copy(keys_hbm.at[idx_v.at[g]], kbuf, ksem).start()

        def wait_gathers(g, ebuf, kbuf, esem, ksem):
            pltpu.make_async_copy(embs_hbm.at[idx_v.at[g]], ebuf, esem).wait()
            pltpu.make_async_copy(keys_hbm.at[idx_v.at[g]], kbuf, ksem).wait()

        def compute_batch(g, buf, kbuf):
            @pl.loop(0, GB, step=2)
            def _t(t):
                tls = [g * GB + t, g * GB + t + 1]
                r0s = [t * K, (t + 1) * K]
                lanes = [lane, lane + K]
                qchs = [[q_v[tl, pl.ds(cc * 16, 16)] for cc in range(4)]
                        for tl in tls]
                for j in range(K):
                    for i in range(2):
                        acc_v[i * K + j, :] = _tree_sum(
                            [qchs[i][cc] * kbuf[r0s[i] + j, pl.ds(cc * 16, 16)]
                             for cc in range(4)])
                attns = []
                for i in range(2):
                    tot = _tree_sum(
                        [plsc.load_gather(acc_v, [lanes[i], cols[d_]])
                         for d_ in range(16)])
                    s = slog_v[tls[i], :] + tot
                    m = jnp.max(s)
                    e = jnp.exp(s - m)
                    e = jnp.where(s > -1e29, e, 0.0)
                    den = jnp.sum(e)
                    den = jnp.where(den > 0.0, den, 1.0)
                    attns.append(e / den)
                ajs = [[attns[i][j] for j in range(K)] for i in range(2)]
                for cc in range(16):
                    for i in range(2):
                        out_v[tls[i], pl.ds(cc * 16, 16)] = _tree_sum(
                            [ajs[i][j] * buf[r0s[i] + j, pl.ds(cc * 16, 16)]
                             for j in range(K)])

        @pl.loop(0, NCHUNK)
        def _chunk(c):
            cbase = pl.multiple_of(wid * CPW + c * CHUNK, CHUNK)
            irow = pl.multiple_of(wid * (CPW * K // GROWS) + c * NBATCH, NBATCH)
            pltpu.sync_copy(idx_hbm.at[pl.ds(irow, NBATCH)], idx_v)
            pltpu.sync_copy(q_hbm.at[pl.ds(cbase, CHUNK)], q_v)
            pltpu.sync_copy(slog_hbm.at[pl.ds(cbase, CHUNK)], slog_v)
            start_gathers(0, bufA, kbufA, semA, semKA)

            @pl.loop(0, NBATCH, step=2)
            def _g(g):
                start_gathers(g + 1, bufB, kbufB, semB, semKB)
                wait_gathers(g, bufA, kbufA, semA, semKA)
                compute_batch(g, bufA, kbufA)

                @pl.when(g + 2 < NBATCH)
                def _():
                    start_gathers(g + 2, bufA, kbufA, semA, semKA)

                wait_gathers(g + 1, bufB, kbufB, semB, semKB)
                compute_batch(g + 1, bufB, kbufB)

            pltpu.sync_copy(out_v, wn_hbm.at[pl.ds(cbase, CHUNK)])

    return sc_kernel(all_embs, keys, idx2, q, slog)


def kernel(center_emb, all_embs, neighbor_indices, neighbor_weights, Wq, Wk, Wg, bg):
    scale = A ** (-0.5)
    wq_scaled = (Wq * scale).astype(jnp.float32)
    wg1 = Wg[:D]
    wg2 = Wg[D:]
    bg2 = bg.reshape(1, D)
    idx2 = neighbor_indices.astype(jnp.int32).reshape(B * K // GROWS, GROWS)

    bb = 2048
    q, slog = pl.pallas_call(
        _tc_pre_body,
        grid=(B // bb,),
        in_specs=[
            pl.BlockSpec((bb, D), lambda i: (i, 0)),
            pl.BlockSpec((bb, K), lambda i: (i, 0)),
            pl.BlockSpec((D, A), lambda i: (0, 0)),
        ],
        out_specs=[
            pl.BlockSpec((bb, A), lambda i: (i, 0)),
            pl.BlockSpec((bb, K), lambda i: (i, 0)),
        ],
        out_shape=[
            jax.ShapeDtypeStruct((B, A), jnp.float32),
            jax.ShapeDtypeStruct((B, K), jnp.float32),
        ],
    )(center_emb, neighbor_weights, wq_scaled)

    nb = 2000
    keys = pl.pallas_call(
        _tc_keys_body,
        grid=(N // nb,),
        in_specs=[
            pl.BlockSpec((nb, D), lambda i: (i, 0)),
            pl.BlockSpec((D, A), lambda i: (0, 0)),
        ],
        out_specs=pl.BlockSpec((nb, 2 * A), lambda i: (i, 0)),
        out_shape=jax.ShapeDtypeStruct((N, 2 * A), jnp.float32),
    )(all_embs, Wk)

    wn = _sc_attention(all_embs, keys, idx2, q, slog)

    out = pl.pallas_call(
        _tc_post_body,
        grid=(B // bb,),
        in_specs=[
            pl.BlockSpec((bb, D), lambda i: (i, 0)),
            pl.BlockSpec((bb, D), lambda i: (i, 0)),
            pl.BlockSpec((D, D), lambda i: (0, 0)),
            pl.BlockSpec((D, D), lambda i: (0, 0)),
            pl.BlockSpec((1, D), lambda i: (0, 0)),
        ],
        out_specs=pl.BlockSpec((bb, D), lambda i: (i, 0)),
        out_shape=jax.ShapeDtypeStruct((B, D), jnp.float32),
    )(center_emb, wn, wg1, wg2, bg2)
    return out
